# baseline (speedup 1.0000x reference)
import jax
import jax.numpy as jnp
from jax.experimental import pallas as pl

B = 65536
HW = 361
BB = 256


def _body(x_ref, p_ref, o_ref):
    x = x_ref[...]
    t = p_ref[...]
    o_ref[:, 0, :] = (x == t).astype(jnp.float32)
    o_ref[:, 1, :] = (x == (1 - t)).astype(jnp.float32)
    o_ref[:, 2, :] = (x == 2).astype(jnp.float32)


def kernel(x, pls):
    xf = x.reshape(B, HW)
    pf = pls.reshape(B, 1)
    out = pl.pallas_call(
        _body,
        grid=(B // BB,),
        in_specs=[
            pl.BlockSpec((BB, HW), lambda i: (i, 0)),
            pl.BlockSpec((BB, 1), lambda i: (i, 0)),
        ],
        out_specs=pl.BlockSpec((BB, 3, HW), lambda i: (i, 0, 0)),
        out_shape=jax.ShapeDtypeStruct((B, 3, HW), jnp.float32),
    )(xf, pf)
    return out.reshape(B, 3, 19, 19)


# speedup vs baseline: 1.0263x; 1.0263x over previous
import jax
import jax.numpy as jnp
from jax.experimental import pallas as pl

B = 65536
HW = 361
BB = 512


def _body(x_ref, p_ref, o_ref):
    x = x_ref[...][:, None, :]
    t0 = p_ref[...][:, :, None]
    ci = jax.lax.broadcasted_iota(jnp.int32, (1, 3, 1), 1)
    tgt = jnp.where(ci == 0, t0, jnp.where(ci == 1, 1 - t0, jnp.full_like(t0, 2)))
    o_ref[...] = (x == tgt).astype(jnp.float32)


def kernel(x, pls):
    xf = x.reshape(B, HW)
    pf = pls.reshape(B, 1)
    out = pl.pallas_call(
        _body,
        grid=(B // BB,),
        in_specs=[
            pl.BlockSpec((BB, HW), lambda i: (i, 0)),
            pl.BlockSpec((BB, 1), lambda i: (i, 0)),
        ],
        out_specs=pl.BlockSpec((BB, 3, HW), lambda i: (i, 0, 0)),
        out_shape=jax.ShapeDtypeStruct((B, 3, HW), jnp.float32),
    )(xf, pf)
    return out.reshape(B, 3, 19, 19)
